# Initial kernel scaffold; baseline (speedup 1.0000x reference)
import jax
import jax.numpy as jnp
from jax.experimental import pallas as pl


def kernel(x, y, y_aux, w):
    raise NotImplementedError("write your pallas kernel here")


# speedup vs baseline: 1.0380x; 1.0380x over previous
import functools

import jax
import jax.numpy as jnp
import numpy as np
from jax.experimental import pallas as pl
from jax.experimental.pallas import tpu as pltpu


@functools.lru_cache(maxsize=None)
def _mix_constants(bs: int):
    with jax.ensure_compile_time_eval():
        key = jax.random.key(42)
        kp, kr, kc = jax.random.split(key, 3)
        perm = jax.random.permutation(kp, bs)
        keep = jax.random.uniform(kr, (bs,)) < 1.0
        perm = jnp.where(keep, perm, jnp.arange(bs))
        coeffs = jax.random.beta(kc, 0.4, 0.4, (bs,)).astype(jnp.float32)
    return np.asarray(perm, dtype=np.int32), np.asarray(coeffs, dtype=np.float32)


def _mix_body(perm_s, coeff_s, xa, xb, ya, yb, za, zb, xo, yo, zo):
    i = pl.program_id(0)
    c = coeff_s[i]
    xo[...] = c * xa[...] + (1.0 - c) * xb[...]
    ym = c * ya[...] + (1.0 - c) * yb[...]
    yo[...] = ym
    zo[...] = jnp.clip(jnp.maximum(za[...], zb[...]) - ym, 0.0, 1.0)


def kernel(x, y, y_aux, w):
    bs = x.shape[0]
    perm, coeffs = _mix_constants(bs)
    n = int(np.prod(x.shape[1:]))
    assert n % 128 == 0
    rows = n // 128
    xr = x.reshape(bs, rows, 128)

    nc = y.shape[1]
    pad = (-(nc + 1)) % 1024
    y2 = jnp.concatenate(
        [y, w[:, None], jnp.zeros((bs, pad), jnp.float32)], axis=1)
    ncp = nc + 1 + pad
    y2r = y2.reshape(bs, ncp // 128, 128)
    yar = jnp.pad(y_aux, ((0, 0), (0, ncp - nc))).reshape(bs, ncp // 128, 128)

    def self_map(i, p, c):
        return (i, 0, 0)

    def perm_map(i, p, c):
        return (p[i], 0, 0)

    xspec = lambda m: pl.BlockSpec((1, rows, 128), m)
    yspec = lambda m: pl.BlockSpec((1, ncp // 128, 128), m)

    grid_spec = pltpu.PrefetchScalarGridSpec(
        num_scalar_prefetch=2,
        grid=(bs,),
        in_specs=[
            xspec(self_map), xspec(perm_map),
            yspec(self_map), yspec(perm_map),
            yspec(self_map), yspec(perm_map),
        ],
        out_specs=[xspec(self_map), yspec(self_map), yspec(self_map)],
    )

    xo, yo, zo = pl.pallas_call(
        _mix_body,
        grid_spec=grid_spec,
        out_shape=[
            jax.ShapeDtypeStruct((bs, rows, 128), jnp.float32),
            jax.ShapeDtypeStruct((bs, ncp // 128, 128), jnp.float32),
            jax.ShapeDtypeStruct((bs, ncp // 128, 128), jnp.float32),
        ],
        compiler_params=pltpu.CompilerParams(
            dimension_semantics=("arbitrary",),
        ),
    )(jnp.asarray(perm), jnp.asarray(coeffs), xr, xr, y2r, y2r, yar, yar)

    x_mix = xo.reshape(x.shape)
    yo2 = yo.reshape(bs, ncp)
    y_mix = yo2[:, :nc]
    w_mix = yo2[:, nc]
    ya_mix = zo.reshape(bs, ncp)[:, :nc]
    return (x_mix, y_mix, ya_mix, w_mix)


# speedup vs baseline: 1.0960x; 1.0559x over previous
import functools

import jax
import jax.numpy as jnp
import numpy as np
from jax.experimental import pallas as pl
from jax.experimental.pallas import tpu as pltpu


@functools.lru_cache(maxsize=None)
def _mix_constants(bs: int):
    with jax.ensure_compile_time_eval():
        key = jax.random.key(42)
        kp, kr, kc = jax.random.split(key, 3)
        perm = jax.random.permutation(kp, bs)
        keep = jax.random.uniform(kr, (bs,)) < 1.0
        perm = jnp.where(keep, perm, jnp.arange(bs))
        coeffs = jax.random.beta(kc, 0.4, 0.4, (bs,)).astype(jnp.float32)
    return np.asarray(perm, dtype=np.int32), np.asarray(coeffs, dtype=np.float32)


@functools.lru_cache(maxsize=None)
def _schedule(bs: int):
    perm, coeffs = _mix_constants(bs)
    visited = np.zeros(bs, dtype=bool)
    order, ishead, isend = [], [], []
    for s in range(bs):
        if visited[s]:
            continue
        i = s
        first = True
        while not visited[i]:
            visited[i] = True
            order.append(i)
            ishead.append(1 if first else 0)
            isend.append(0)
            first = False
            i = int(perm[i])
        isend[-1] = 1
    order = np.asarray(order, dtype=np.int32)
    ishead = np.asarray(ishead, dtype=np.int32)
    isend = np.asarray(isend, dtype=np.int32)

    ld = np.concatenate([order, order[-1:]])
    oidx = np.concatenate([order[:1], order])
    bidx = perm[oidx]
    e = np.concatenate([np.zeros(1, np.int32), isend])
    hd = np.concatenate([ishead, np.zeros(1, np.int32)])
    cs = coeffs[oidx]
    return ld, oidx, bidx, e, hd, cs


def _mix_body(ld, oidx, bidx, e, hd, cs,
              xin, ya, yb, za, zb, xo, yo, zo, prev, head):
    t = pl.program_id(0)
    c = cs[t]
    rowb = jnp.where(e[t] == 1, head[...], xin[...])
    xo[...] = c * prev[...] + (1.0 - c) * rowb
    ym = c * ya[...] + (1.0 - c) * yb[...]
    yo[...] = ym
    zo[...] = jnp.clip(jnp.maximum(za[...], zb[...]) - ym, 0.0, 1.0)

    @pl.when(hd[t] == 1)
    def _():
        head[...] = xin[...]

    prev[...] = xin[...]


def kernel(x, y, y_aux, w):
    bs = x.shape[0]
    ld, oidx, bidx, e, hd, cs = _schedule(bs)
    n = int(np.prod(x.shape[1:]))
    assert n % 128 == 0
    rows = n // 128
    xr = x.reshape(bs, rows, 128)

    nc = y.shape[1]
    pad = (-(nc + 1)) % 1024
    y2 = jnp.concatenate(
        [y, w[:, None], jnp.zeros((bs, pad), jnp.float32)], axis=1)
    ncp = nc + 1 + pad
    y2r = y2.reshape(bs, ncp // 128, 128)
    yar = jnp.pad(y_aux, ((0, 0), (0, ncp - nc))).reshape(bs, ncp // 128, 128)

    def ld_map(t, ld, oidx, bidx, e, hd, cs):
        return (ld[t], 0, 0)

    def o_map(t, ld, oidx, bidx, e, hd, cs):
        return (oidx[t], 0, 0)

    def b_map(t, ld, oidx, bidx, e, hd, cs):
        return (bidx[t], 0, 0)

    xspec = lambda m: pl.BlockSpec((1, rows, 128), m)
    yspec = lambda m: pl.BlockSpec((1, ncp // 128, 128), m)

    grid_spec = pltpu.PrefetchScalarGridSpec(
        num_scalar_prefetch=6,
        grid=(bs + 1,),
        in_specs=[
            xspec(ld_map),
            yspec(o_map), yspec(b_map),
            yspec(o_map), yspec(b_map),
        ],
        out_specs=[xspec(o_map), yspec(o_map), yspec(o_map)],
        scratch_shapes=[
            pltpu.VMEM((1, rows, 128), jnp.float32),
            pltpu.VMEM((1, rows, 128), jnp.float32),
        ],
    )

    xo, yo, zo = pl.pallas_call(
        _mix_body,
        grid_spec=grid_spec,
        out_shape=[
            jax.ShapeDtypeStruct((bs, rows, 128), jnp.float32),
            jax.ShapeDtypeStruct((bs, ncp // 128, 128), jnp.float32),
            jax.ShapeDtypeStruct((bs, ncp // 128, 128), jnp.float32),
        ],
        compiler_params=pltpu.CompilerParams(
            dimension_semantics=("arbitrary",),
        ),
    )(jnp.asarray(ld), jnp.asarray(oidx), jnp.asarray(bidx),
      jnp.asarray(e), jnp.asarray(hd), jnp.asarray(cs),
      xr, y2r, y2r, yar, yar)

    x_mix = xo.reshape(x.shape)
    yo2 = yo.reshape(bs, ncp)
    y_mix = yo2[:, :nc]
    w_mix = yo2[:, nc]
    ya_mix = zo.reshape(bs, ncp)[:, :nc]
    return (x_mix, y_mix, ya_mix, w_mix)


# speedup vs baseline: 1.1035x; 1.0068x over previous
import functools

import jax
import jax.numpy as jnp
import numpy as np
from jax.experimental import pallas as pl
from jax.experimental.pallas import tpu as pltpu


@functools.lru_cache(maxsize=None)
def _mix_constants(bs: int):
    with jax.ensure_compile_time_eval():
        key = jax.random.key(42)
        kp, kr, kc = jax.random.split(key, 3)
        perm = jax.random.permutation(kp, bs)
        keep = jax.random.uniform(kr, (bs,)) < 1.0
        perm = jnp.where(keep, perm, jnp.arange(bs))
        coeffs = jax.random.beta(kc, 0.4, 0.4, (bs,)).astype(jnp.float32)
    return np.asarray(perm, dtype=np.int32), np.asarray(coeffs, dtype=np.float32)


@functools.lru_cache(maxsize=None)
def _schedule(bs: int):
    perm, coeffs = _mix_constants(bs)
    visited = np.zeros(bs, dtype=bool)
    order, ishead, isend = [], [], []
    for s in range(bs):
        if visited[s]:
            continue
        i = s
        first = True
        while not visited[i]:
            visited[i] = True
            order.append(i)
            ishead.append(1 if first else 0)
            isend.append(0)
            first = False
            i = int(perm[i])
        isend[-1] = 1
    order = np.asarray(order, dtype=np.int32)
    ishead = np.asarray(ishead, dtype=np.int32)
    isend = np.asarray(isend, dtype=np.int32)

    n = bs + 1
    la = np.empty(n, np.int32)
    lb = np.empty(n, np.int32)
    la[0] = order[0]
    lb[0] = order[1] if bs > 1 else order[0]
    for t in range(1, bs):
        if t % 2 == 1:
            lb[t] = order[t]
            la[t] = la[t - 1]
        else:
            la[t] = order[t]
            lb[t] = lb[t - 1]
    la[bs] = la[bs - 1]
    lb[bs] = lb[bs - 1]

    oidx = np.concatenate([order[:1], order])
    bidx = perm[oidx]
    e = np.concatenate([np.zeros(1, np.int32), isend])
    hd = np.concatenate([ishead, np.zeros(1, np.int32)])
    cs = coeffs[oidx]
    return la, lb, oidx, bidx, e, hd, cs


def _mix_body(la, lb, oidx, bidx, e, hd, cs,
              xa, xb, y2f, yaf, xo, yof, zof, head):
    t = pl.program_id(0)
    c = cs[t]
    even = t % 2 == 0
    end = e[t] == 1

    @pl.when(jnp.logical_and(even, jnp.logical_not(end)))
    def _():
        xo[...] = c * xb[...] + (1.0 - c) * xa[...]

    @pl.when(jnp.logical_and(jnp.logical_not(even), jnp.logical_not(end)))
    def _():
        xo[...] = c * xa[...] + (1.0 - c) * xb[...]

    @pl.when(jnp.logical_and(even, end))
    def _():
        xo[...] = c * xb[...] + (1.0 - c) * head[...]

    @pl.when(jnp.logical_and(jnp.logical_not(even), end))
    def _():
        xo[...] = c * xa[...] + (1.0 - c) * head[...]

    @pl.when(jnp.logical_and(hd[t] == 1, even))
    def _():
        head[...] = xa[...]

    @pl.when(jnp.logical_and(hd[t] == 1, jnp.logical_not(even)))
    def _():
        head[...] = xb[...]

    o = oidx[t]
    b = bidx[t]
    ym = c * y2f[o] + (1.0 - c) * y2f[b]
    yof[o] = ym
    zof[o] = jnp.clip(jnp.maximum(yaf[o], yaf[b]) - ym, 0.0, 1.0)


def kernel(x, y, y_aux, w):
    bs = x.shape[0]
    la, lb, oidx, bidx, e, hd, cs = _schedule(bs)
    n = int(np.prod(x.shape[1:]))
    assert n % 128 == 0
    rows = n // 128
    xr = x.reshape(bs, rows, 128)

    nc = y.shape[1]
    pad = (-(nc + 1)) % 1024
    y2 = jnp.concatenate(
        [y, w[:, None], jnp.zeros((bs, pad), jnp.float32)], axis=1)
    ncp = nc + 1 + pad
    y2r = y2.reshape(bs, ncp // 128, 128)
    yar = jnp.pad(y_aux, ((0, 0), (0, ncp - nc))).reshape(bs, ncp // 128, 128)

    def a_map(t, la, lb, oidx, bidx, e, hd, cs):
        return (la[t], 0, 0)

    def b_map(t, la, lb, oidx, bidx, e, hd, cs):
        return (lb[t], 0, 0)

    def o_map(t, la, lb, oidx, bidx, e, hd, cs):
        return (oidx[t], 0, 0)

    def full_map(t, la, lb, oidx, bidx, e, hd, cs):
        return (0, 0, 0)

    xspec = lambda m: pl.BlockSpec((1, rows, 128), m)
    yfull = pl.BlockSpec((bs, ncp // 128, 128), full_map)

    grid_spec = pltpu.PrefetchScalarGridSpec(
        num_scalar_prefetch=7,
        grid=(bs + 1,),
        in_specs=[xspec(a_map), xspec(b_map), yfull, yfull],
        out_specs=[xspec(o_map), yfull, yfull],
        scratch_shapes=[
            pltpu.VMEM((1, rows, 128), jnp.float32),
        ],
    )

    xo, yo, zo = pl.pallas_call(
        _mix_body,
        grid_spec=grid_spec,
        out_shape=[
            jax.ShapeDtypeStruct((bs, rows, 128), jnp.float32),
            jax.ShapeDtypeStruct((bs, ncp // 128, 128), jnp.float32),
            jax.ShapeDtypeStruct((bs, ncp // 128, 128), jnp.float32),
        ],
        compiler_params=pltpu.CompilerParams(
            dimension_semantics=("arbitrary",),
        ),
    )(jnp.asarray(la), jnp.asarray(lb), jnp.asarray(oidx), jnp.asarray(bidx),
      jnp.asarray(e), jnp.asarray(hd), jnp.asarray(cs),
      xr, xr, y2r, yar)

    x_mix = xo.reshape(x.shape)
    yo2 = yo.reshape(bs, ncp)
    y_mix = yo2[:, :nc]
    w_mix = yo2[:, nc]
    ya_mix = zo.reshape(bs, ncp)[:, :nc]
    return (x_mix, y_mix, ya_mix, w_mix)
